# baseline (speedup 1.0000x reference)
import jax
import jax.numpy as jnp
from jax import lax
from jax.experimental import pallas as pl
from jax.experimental.pallas import tpu as pltpu
from jax.experimental.pallas import tpu_sc as plsc

E = 64
K = 8
T = 32768
NC = 2
NS = 16
NW = NC * NS
TOK = T // NW
PAIRS = TOK // 2
TT = T // 128
TPW = TT // NW
ROWS = 8 * TPW * 8
RSTRIDE = 136
XLEN = (ROWS - 1) * RSTRIDE + 63 + 128 + 57


def _body(x4_ref, bias_ref, idx_out_ref, w_out_ref, stage_v, x_v, bias_v,
          iout_v, wout_v):
  wid = lax.axis_index("c") * NS + lax.axis_index("s")
  c0 = wid * TPW
  pltpu.sync_copy(bias_ref, bias_v)

  iota = lax.iota(jnp.int32, 16)
  mask8 = iota < 8
  shift8 = (iota + 8) & 15
  full7 = jnp.full((16,), 7, jnp.int32)
  full15 = jnp.full((16,), 15, jnp.int32)
  lane_div8 = iota >> 3
  lane_mod8 = iota & 7
  bias_c = [bias_v[pl.ds(16 * j, 16)] for j in range(4)]
  idx_c = [iota + 16 * j for j in range(4)]
  iota64 = iota * 64

  for half in range(2):
    for r in range(8):
      pltpu.sync_copy(x4_ref.at[r].at[pl.ds(c0 + half * (TPW // 2),
                                            TPW // 2)], stage_v.at[r])

    def _transpose(cc, carry, half=half):
      tbase = ((half * (TPW // 2) + cc) * 128) * 64
      for r in range(8):
        for er in range(8):
          for m in range(8):
            v = stage_v[r, cc, er, pl.ds(16 * m, 16)]
            plsc.store_scatter(
                x_v, [iota64 + (tbase + (16 * m) * 64 + 8 * r + er)], v)
      return carry

    lax.fori_loop(0, TPW // 2, _transpose, 0)

  def combine(uv, ui, vv, vi):
    vvs = jnp.take_along_axis(vv, shift8, axis=0)
    vis = jnp.take_along_axis(vi, shift8, axis=0)
    return jnp.where(mask8, uv, vvs), jnp.where(mask8, ui, vis)

  def top8(off):
    s = []
    for j in range(4):
      c = x_v[pl.ds(off + 16 * j, 16)]
      s.append(plsc.sort_key_val(c + bias_c[j], idx_c[j], descending=True))
    xv, xi = combine(s[0][0], s[0][1], s[1][0], s[1][1])
    yv, yi = combine(s[2][0], s[2][1], s[3][0], s[3][1])
    xv, xi = plsc.sort_key_val(xv, xi, descending=True)
    yv, yi = plsc.sort_key_val(yv, yi, descending=True)
    zv, zi = combine(xv, xi, yv, yi)
    return plsc.sort_key_val(zv, zi, descending=True)

  @plsc.parallel_loop(0, PAIRS, 1, unroll=4)
  def _pair_loop(p):
    tloc = p >> 6
    l0 = (p & 63) * 2
    t_vec = jnp.full((16,), 0, jnp.int32) + tloc
    av, ai = top8(p * 128)
    bv, bi = top8(p * 128 + 64)
    wv, wi = combine(av, ai, bv, bi)
    e = jnp.exp(wv)
    c = plsc.cumsum(e)
    g7 = jnp.take_along_axis(c, full7, axis=0)
    g15 = jnp.take_along_axis(c, full15, axis=0)
    denom = jnp.where(mask8, g7, g15 - g7)
    lanes = jnp.full((16,), 0, jnp.int32) + l0 + lane_div8
    plsc.store_scatter(iout_v, [t_vec, lane_mod8, lanes], wi)
    plsc.store_scatter(wout_v, [t_vec, lane_mod8, lanes], e / denom)

  pltpu.sync_copy(iout_v, idx_out_ref.at[pl.ds(c0, TPW)])
  pltpu.sync_copy(wout_v, w_out_ref.at[pl.ds(c0, TPW)])


_router = pl.kernel(
    _body,
    out_type=(
        jax.ShapeDtypeStruct((TT, K, 128), jnp.int32),
        jax.ShapeDtypeStruct((TT, K, 128), jnp.float32),
    ),
    mesh=plsc.VectorSubcoreMesh(
        core_axis_name="c", subcore_axis_name="s", num_cores=NC,
        num_subcores=NS),
    compiler_params=pltpu.CompilerParams(needs_layout_passes=False),
    scratch_types=[
        pltpu.VMEM((8, TPW // 2, 8, 128), jnp.float32),
        pltpu.VMEM((TOK * E,), jnp.float32),
        pltpu.VMEM((E,), jnp.float32),
        pltpu.VMEM((TPW, K, 128), jnp.int32),
        pltpu.VMEM((TPW, K, 128), jnp.float32),
    ],
)


def kernel(gate_logits, bias):
  x4 = gate_logits.T.reshape(8, 8, TT, 128).transpose(0, 2, 1, 3)
  idx3, w3 = _router(x4, bias)
  idx = idx3.transpose(0, 2, 1).reshape(T, K)
  w = w3.transpose(0, 2, 1).reshape(T, K)
  return idx, w


# speedup vs baseline: 1.4139x; 1.4139x over previous
import jax
import jax.numpy as jnp
from jax import lax
from jax.experimental import pallas as pl
from jax.experimental.pallas import tpu as pltpu
from jax.experimental.pallas import tpu_sc as plsc

E = 64
K = 8
T = 32768
NC = 2
NS = 16
NW = NC * NS
TOK = T // NW
PAIRS = TOK // 2
TT = T // 128
TPW = TT // NW
ROWS = 8 * TPW * 8
RSTRIDE = 136
XLEN = (ROWS - 1) * RSTRIDE + 63 + 128 + 57


def _body(x4_ref, bias_ref, idx_out_ref, w_out_ref, stage_v, x_v, bias_v,
          iout_v, wout_v):
  wid = lax.axis_index("c") * NS + lax.axis_index("s")
  c0 = wid * TPW
  pltpu.sync_copy(bias_ref, bias_v)

  iota = lax.iota(jnp.int32, 16)
  mask8 = iota < 8
  shift8 = (iota + 8) & 15
  full7 = jnp.full((16,), 7, jnp.int32)
  full15 = jnp.full((16,), 15, jnp.int32)
  lane_div8 = iota >> 3
  lane_mod8 = iota & 7
  bias_c = [bias_v[pl.ds(16 * j, 16)] for j in range(4)]
  idx_c = [iota + 16 * j for j in range(4)]
  iota65 = iota * 65

  for half in range(2):
    for r in range(8):
      pltpu.sync_copy(x4_ref.at[r].at[pl.ds(c0 + half * (TPW // 2),
                                            TPW // 2)], stage_v.at[r])

    def _transpose(cc, carry, half=half):
      tbase = ((half * (TPW // 2) + cc) * 128) * 65
      for r in range(8):
        for er in range(8):
          for m in range(8):
            v = stage_v[r, cc, er, pl.ds(16 * m, 16)]
            plsc.store_scatter(
                x_v, [iota65 + (tbase + (16 * m) * 65 + 8 * r + er)], v)
      return carry

    lax.fori_loop(0, TPW // 2, _transpose, 0)

  def combine(uv, ui, vv, vi):
    vvs = jnp.take_along_axis(vv, shift8, axis=0)
    vis = jnp.take_along_axis(vi, shift8, axis=0)
    return jnp.where(mask8, uv, vvs), jnp.where(mask8, ui, vis)

  def top8(off):
    s = []
    for j in range(4):
      c = x_v[pl.ds(off + 16 * j, 16)]
      s.append(plsc.sort_key_val(c + bias_c[j], idx_c[j], descending=True))
    xv, xi = combine(s[0][0], s[0][1], s[1][0], s[1][1])
    yv, yi = combine(s[2][0], s[2][1], s[3][0], s[3][1])
    xv, xi = plsc.sort_key_val(xv, xi, descending=True)
    yv, yi = plsc.sort_key_val(yv, yi, descending=True)
    zv, zi = combine(xv, xi, yv, yi)
    return plsc.sort_key_val(zv, zi, descending=True)

  @plsc.parallel_loop(0, PAIRS, 1, unroll=4)
  def _pair_loop(p):
    tloc = p >> 6
    l0 = (p & 63) * 2
    t_vec = jnp.full((16,), 0, jnp.int32) + tloc
    av, ai = top8(p * 130)
    bv, bi = top8(p * 130 + 65)
    wv, wi = combine(av, ai, bv, bi)
    e = jnp.exp(wv)
    c = plsc.cumsum(e)
    g7 = jnp.take_along_axis(c, full7, axis=0)
    g15 = jnp.take_along_axis(c, full15, axis=0)
    denom = jnp.where(mask8, g7, g15 - g7)
    lanes = jnp.full((16,), 0, jnp.int32) + l0 + lane_div8
    plsc.store_scatter(iout_v, [t_vec, lane_mod8, lanes], wi)
    plsc.store_scatter(wout_v, [t_vec, lane_mod8, lanes], e / denom)

  pltpu.sync_copy(iout_v, idx_out_ref.at[pl.ds(c0, TPW)])
  pltpu.sync_copy(wout_v, w_out_ref.at[pl.ds(c0, TPW)])


_router = pl.kernel(
    _body,
    out_type=(
        jax.ShapeDtypeStruct((TT, K, 128), jnp.int32),
        jax.ShapeDtypeStruct((TT, K, 128), jnp.float32),
    ),
    mesh=plsc.VectorSubcoreMesh(
        core_axis_name="c", subcore_axis_name="s", num_cores=NC,
        num_subcores=NS),
    compiler_params=pltpu.CompilerParams(needs_layout_passes=False),
    scratch_types=[
        pltpu.VMEM((8, TPW // 2, 8, 128), jnp.float32),
        pltpu.VMEM((TOK * 65,), jnp.float32),
        pltpu.VMEM((E,), jnp.float32),
        pltpu.VMEM((TPW, K, 128), jnp.int32),
        pltpu.VMEM((TPW, K, 128), jnp.float32),
    ],
)


def kernel(gate_logits, bias):
  x4 = gate_logits.T.reshape(8, 8, TT, 128).transpose(0, 2, 1, 3)
  idx3, w3 = _router(x4, bias)
  idx = idx3.transpose(0, 2, 1).reshape(T, K)
  w = w3.transpose(0, 2, 1).reshape(T, K)
  return idx, w


# speedup vs baseline: 1.5681x; 1.1091x over previous
import jax
import jax.numpy as jnp
from jax import lax
from jax.experimental import pallas as pl
from jax.experimental.pallas import tpu as pltpu
from jax.experimental.pallas import tpu_sc as plsc

E = 64
K = 8
T = 32768
NC = 2
NS = 16
NW = NC * NS
TOK = T // NW
PAIRS = TOK // 2
TT = T // 128
TPW = TT // NW
ROWS = 8 * TPW * 8
RSTRIDE = 136
XLEN = (ROWS - 1) * RSTRIDE + 63 + 128 + 57


def _body(x4_ref, bias_ref, idx_out_ref, w_out_ref, stage_v, x_v, bias_v,
          iout_v, wout_v, sem0, sem1):
  wid = lax.axis_index("c") * NS + lax.axis_index("s")
  c0 = wid * TPW
  pltpu.sync_copy(bias_ref, bias_v)

  iota = lax.iota(jnp.int32, 16)
  mask8 = iota < 8
  shift8 = (iota + 8) & 15
  full7 = jnp.full((16,), 7, jnp.int32)
  full15 = jnp.full((16,), 15, jnp.int32)
  lane_div8 = iota >> 3
  lane_mod8 = iota & 7
  bias_c = [bias_v[pl.ds(16 * j, 16)] for j in range(4)]
  idx_c = [iota + 16 * j for j in range(4)]
  iota65 = iota * 65

  sems = (sem0, sem1)
  pend = [pltpu.async_copy(x4_ref.at[r].at[pl.ds(c0, 2)],
                           stage_v.at[0].at[r], sem0) for r in range(8)]
  for q in range(4):
    buf = q % 2
    for cp in pend:
      cp.wait()
    if q < 3:
      nbuf = (q + 1) % 2
      pend = [pltpu.async_copy(x4_ref.at[r].at[pl.ds(c0 + (q + 1) * 2, 2)],
                               stage_v.at[nbuf].at[r], sems[nbuf])
              for r in range(8)]

    def _transpose(cc, carry2, q=q, buf=buf):
      tbase = ((q * 2 + cc) * 128) * 65
      for r in range(8):
        for er in range(8):
          for m in range(8):
            v = stage_v[buf, r, cc, er, pl.ds(16 * m, 16)]
            plsc.store_scatter(
                x_v, [iota65 + (tbase + (16 * m) * 65 + 8 * r + er)], v)
      return carry2

    lax.fori_loop(0, 2, _transpose, 0)

  def combine(uv, ui, vv, vi):
    vvs = jnp.take_along_axis(vv, shift8, axis=0)
    vis = jnp.take_along_axis(vi, shift8, axis=0)
    return jnp.where(mask8, uv, vvs), jnp.where(mask8, ui, vis)

  def top8(off):
    s = []
    for j in range(4):
      c = x_v[pl.ds(off + 16 * j, 16)]
      s.append(plsc.sort_key_val(c + bias_c[j], idx_c[j], descending=True))
    xv, xi = combine(s[0][0], s[0][1], s[1][0], s[1][1])
    yv, yi = combine(s[2][0], s[2][1], s[3][0], s[3][1])
    xv, xi = plsc.sort_key_val(xv, xi, descending=True)
    yv, yi = plsc.sort_key_val(yv, yi, descending=True)
    zv, zi = combine(xv, xi, yv, yi)
    return plsc.sort_key_val(zv, zi, descending=True)

  @plsc.parallel_loop(0, PAIRS, 1, unroll=4)
  def _pair_loop(p):
    tloc = p >> 6
    l0 = (p & 63) * 2
    t_vec = jnp.full((16,), 0, jnp.int32) + tloc
    av, ai = top8(p * 130)
    bv, bi = top8(p * 130 + 65)
    wv, wi = combine(av, ai, bv, bi)
    e = jnp.exp(wv)
    c = plsc.cumsum(e)
    g7 = jnp.take_along_axis(c, full7, axis=0)
    g15 = jnp.take_along_axis(c, full15, axis=0)
    denom = jnp.where(mask8, g7, g15 - g7)
    lanes = jnp.full((16,), 0, jnp.int32) + l0 + lane_div8
    plsc.store_scatter(iout_v, [t_vec, lane_mod8, lanes], wi)
    plsc.store_scatter(wout_v, [t_vec, lane_mod8, lanes], e / denom)

  pltpu.sync_copy(iout_v, idx_out_ref.at[pl.ds(c0, TPW)])
  pltpu.sync_copy(wout_v, w_out_ref.at[pl.ds(c0, TPW)])


_router = pl.kernel(
    _body,
    out_type=(
        jax.ShapeDtypeStruct((TT, K, 128), jnp.int32),
        jax.ShapeDtypeStruct((TT, K, 128), jnp.float32),
    ),
    mesh=plsc.VectorSubcoreMesh(
        core_axis_name="c", subcore_axis_name="s", num_cores=NC,
        num_subcores=NS),
    compiler_params=pltpu.CompilerParams(needs_layout_passes=False),
    scratch_types=[
        pltpu.VMEM((2, 8, 2, 8, 128), jnp.float32),
        pltpu.VMEM((TOK * 65,), jnp.float32),
        pltpu.VMEM((E,), jnp.float32),
        pltpu.VMEM((TPW, K, 128), jnp.int32),
        pltpu.VMEM((TPW, K, 128), jnp.float32),
        pltpu.SemaphoreType.DMA,
        pltpu.SemaphoreType.DMA,
    ],
)


def kernel(gate_logits, bias):
  x4 = gate_logits.T.reshape(8, 8, TT, 128).transpose(0, 2, 1, 3)
  idx3, w3 = _router(x4, bias)
  idx = idx3.transpose(0, 2, 1).reshape(T, K)
  w = w3.transpose(0, 2, 1).reshape(T, K)
  return idx, w
